# baseline (speedup 1.0000x reference)
import functools

import jax
import jax.numpy as jnp
from jax import lax
from jax.experimental import pallas as pl
from jax.experimental.pallas import tpu as pltpu
from jax.experimental.pallas import tpu_sc as plsc

B = 4096
V = 100000
T = 200
NW = 32
CHUNK = (B * T) // NW
LANES = 16
SW = 128
G = 10
NG = CHUNK // (G * SW)
DEPTH = 4
GEL = G * SW
GCH = GEL // LANES
SCALE = -1.0 / float(B * T)


def _mesh():
    return plsc.VectorSubcoreMesh(core_axis_name="c", subcore_axis_name="s")


@functools.partial(
    pl.kernel,
    mesh=_mesh(),
    out_type=jax.ShapeDtypeStruct((NW, LANES), jnp.float32),
    scratch_types=[
        pltpu.VMEM((CHUNK // SW, SW), jnp.int32),
        pltpu.VMEM((CHUNK,), jnp.int32),
        pltpu.VMEM((CHUNK,), jnp.float32),
        pltpu.VMEM((LANES,), jnp.float32),
        pltpu.SemaphoreType.DMA,
        pltpu.SemaphoreType.DMA,
        pltpu.SemaphoreType.DMA,
        pltpu.SemaphoreType.DMA,
        pltpu.SemaphoreType.DMA,
    ],
)
def _sc_gather_loss(tags_hbm, flat_hbm, out_hbm,
                    lbl_v, idx_v, val_v, part_v,
                    sem_a, sem_b, sem_c, sem_d, sem_l):
    nc = 2
    wid = lax.axis_index("s") * nc + lax.axis_index("c")

    def lstage(m, c):
        pltpu.make_async_copy(tags_hbm.at[m, wid],
                              lbl_v.at[pl.ds(m * 8, 8), :], sem_l).start()
        return c

    lax.fori_loop(0, T // 8, lstage, 0)

    def lwait(m, c):
        pltpu.make_async_copy(tags_hbm.at[0, 0],
                              lbl_v.at[pl.ds(0, 8), :], sem_l).wait()
        return c

    lax.fori_loop(0, T // 8, lwait, 0)

    wbase = wid * 1024
    lane = lax.iota(jnp.int32, 16)

    def cbody(i, c):
        sl = pl.ds(i * LANES, LANES)
        q0 = (i & 7) << 4
        v = lbl_v[i >> 3, pl.ds(q0, LANES)]
        idx_v[sl] = ((v >> 3) << 15) + ((v & 7) << 7) + (wbase + q0 + lane)
        return c

    def compute(g):
        lax.fori_loop(g * GCH, (g + 1) * GCH, cbody, 0, unroll=4)

    def fire(g, sem):
        def fbody(t, c):
            sl = pl.ds(t * SW, SW)
            pltpu.make_async_copy(flat_hbm.at[idx_v.at[sl]], val_v.at[sl],
                                  sem).start()
            return c
        lax.fori_loop(g * G, (g + 1) * G, fbody, 0)

    def drain(g, sem):
        sl = pl.ds(g * GEL, GEL)
        pltpu.make_async_copy(flat_hbm.at[idx_v.at[sl]], val_v.at[sl],
                              sem).wait()

    def rbody(i, acc):
        sl = pl.ds(i * LANES, LANES)
        v = lbl_v[i >> 3, pl.ds((i & 7) << 4, LANES)]
        return acc + jnp.where(v != 0, val_v[sl], 0.0)

    def reduce(g, acc):
        return lax.fori_loop(g * GCH, (g + 1) * GCH, rbody, acc, unroll=4)

    def fused_body(gr, gc):
        def fbody(i, acc):
            slr = pl.ds(gr * GEL + i * LANES, LANES)
            ir = gr * GCH + i
            vr = lbl_v[ir >> 3, pl.ds((ir & 7) << 4, LANES)]
            acc = acc + jnp.where(vr != 0, val_v[slr], 0.0)
            ic = gc * GCH + i
            slc = pl.ds(gc * GEL + i * LANES, LANES)
            q0 = (ic & 7) << 4
            v = lbl_v[ic >> 3, pl.ds(q0, LANES)]
            idx_v[slc] = ((v >> 3) << 15) + ((v & 7) << 7) + (wbase + q0 + lane)
            return acc
        return fbody

    sems = (sem_a, sem_b, sem_c, sem_d, sem_l)
    nsem = len(sems)
    acc = jnp.zeros((LANES,), jnp.float32)
    for g in range(DEPTH):
        compute(g)
        fire(g, sems[g])
    for g in range(NG):
        drain(g, sems[g % nsem])
        if g + DEPTH < NG:
            acc = lax.fori_loop(0, GCH, fused_body(g, g + DEPTH), acc,
                                unroll=4)
            fire(g + DEPTH, sems[(g + DEPTH) % nsem])
        else:
            acc = reduce(g, acc)

    part_v[...] = acc * SCALE
    pltpu.sync_copy(part_v, out_hbm.at[wid])


def kernel(log_prob, tags_label):
    flat = (log_prob.T
            .reshape(V // 8, 8, B // 128, 128)
            .transpose(0, 2, 1, 3)
            .reshape(-1))
    tags = (tags_label.astype(jnp.int32).T
            .reshape(T // 8, 8, B // 128, 128)
            .transpose(0, 2, 1, 3))
    partials = _sc_gather_loss(tags, flat)
    return jnp.sum(partials)


# speedup vs baseline: 1.0085x; 1.0085x over previous
import functools

import jax
import jax.numpy as jnp
from jax import lax
from jax.experimental import pallas as pl
from jax.experimental.pallas import tpu as pltpu
from jax.experimental.pallas import tpu_sc as plsc

B = 4096
V = 100000
T = 200
NW = 32
CHUNK = (B * T) // NW
LANES = 16
SW = 128
G = 20
NG = CHUNK // (G * SW)
DEPTH = 4
GEL = G * SW
GCH = GEL // LANES
SCALE = -1.0 / float(B * T)


def _mesh():
    return plsc.VectorSubcoreMesh(core_axis_name="c", subcore_axis_name="s")


@functools.partial(
    pl.kernel,
    mesh=_mesh(),
    out_type=jax.ShapeDtypeStruct((NW, LANES), jnp.float32),
    scratch_types=[
        pltpu.VMEM((CHUNK // SW, SW), jnp.int32),
        pltpu.VMEM((CHUNK,), jnp.int32),
        pltpu.VMEM((CHUNK,), jnp.float32),
        pltpu.VMEM((LANES,), jnp.float32),
        pltpu.SemaphoreType.DMA,
        pltpu.SemaphoreType.DMA,
        pltpu.SemaphoreType.DMA,
        pltpu.SemaphoreType.DMA,
        pltpu.SemaphoreType.DMA,
    ],
)
def _sc_gather_loss(tags_hbm, flat_hbm, out_hbm,
                    lbl_v, idx_v, val_v, part_v,
                    sem_a, sem_b, sem_c, sem_d, sem_l):
    nc = 2
    wid = lax.axis_index("s") * nc + lax.axis_index("c")

    def lstage(m, c):
        pltpu.make_async_copy(tags_hbm.at[m, wid],
                              lbl_v.at[pl.ds(m * 8, 8), :], sem_l).start()
        return c

    lax.fori_loop(0, T // 8, lstage, 0)

    def lwait(m, c):
        pltpu.make_async_copy(tags_hbm.at[0, 0],
                              lbl_v.at[pl.ds(0, 8), :], sem_l).wait()
        return c

    lax.fori_loop(0, T // 8, lwait, 0)

    wbase = wid * 1024
    lane = lax.iota(jnp.int32, 16)

    def cbody(i, c):
        sl = pl.ds(i * LANES, LANES)
        q0 = (i & 7) << 4
        v = lbl_v[i >> 3, pl.ds(q0, LANES)]
        idx_v[sl] = ((v >> 3) << 15) + ((v & 7) << 7) + (wbase + q0 + lane)
        return c

    def compute(g):
        lax.fori_loop(g * GCH, (g + 1) * GCH, cbody, 0, unroll=4)

    def fire(g, sem):
        def fbody(t, c):
            sl = pl.ds(t * SW, SW)
            pltpu.make_async_copy(flat_hbm.at[idx_v.at[sl]], val_v.at[sl],
                                  sem).start()
            return c
        lax.fori_loop(g * G, (g + 1) * G, fbody, 0)

    def drain(g, sem):
        sl = pl.ds(g * GEL, GEL)
        pltpu.make_async_copy(flat_hbm.at[idx_v.at[sl]], val_v.at[sl],
                              sem).wait()

    def rbody(i, acc):
        sl = pl.ds(i * LANES, LANES)
        v = lbl_v[i >> 3, pl.ds((i & 7) << 4, LANES)]
        return acc + jnp.where(v != 0, val_v[sl], 0.0)

    def reduce(g, acc):
        return lax.fori_loop(g * GCH, (g + 1) * GCH, rbody, acc, unroll=4)

    def fused_body(gr, gc):
        def fbody(i, acc):
            slr = pl.ds(gr * GEL + i * LANES, LANES)
            ir = gr * GCH + i
            vr = lbl_v[ir >> 3, pl.ds((ir & 7) << 4, LANES)]
            acc = acc + jnp.where(vr != 0, val_v[slr], 0.0)
            ic = gc * GCH + i
            slc = pl.ds(gc * GEL + i * LANES, LANES)
            q0 = (ic & 7) << 4
            v = lbl_v[ic >> 3, pl.ds(q0, LANES)]
            idx_v[slc] = ((v >> 3) << 15) + ((v & 7) << 7) + (wbase + q0 + lane)
            return acc
        return fbody

    sems = (sem_a, sem_b, sem_c, sem_d, sem_l)
    nsem = len(sems)
    acc = jnp.zeros((LANES,), jnp.float32)
    for g in range(DEPTH):
        compute(g)
        fire(g, sems[g])
    for g in range(NG):
        drain(g, sems[g % nsem])
        if g + DEPTH < NG:
            acc = lax.fori_loop(0, GCH, fused_body(g, g + DEPTH), acc,
                                unroll=8)
            fire(g + DEPTH, sems[(g + DEPTH) % nsem])
        else:
            acc = reduce(g, acc)

    part_v[...] = acc * SCALE
    pltpu.sync_copy(part_v, out_hbm.at[wid])


def kernel(log_prob, tags_label):
    flat = (log_prob.T
            .reshape(V // 8, 8, B // 128, 128)
            .transpose(0, 2, 1, 3)
            .reshape(-1))
    tags = (tags_label.astype(jnp.int32).T
            .reshape(T // 8, 8, B // 128, 128)
            .transpose(0, 2, 1, 3))
    partials = _sc_gather_loss(tags, flat)
    return jnp.sum(partials)


# speedup vs baseline: 1.0190x; 1.0104x over previous
import functools

import jax
import jax.numpy as jnp
from jax import lax
from jax.experimental import pallas as pl
from jax.experimental.pallas import tpu as pltpu
from jax.experimental.pallas import tpu_sc as plsc

B = 4096
V = 100000
T = 200
NW = 32
CHUNK = (B * T) // NW
LANES = 16
SW = 128
G = 20
NG = CHUNK // (G * SW)
DEPTH = 4
GEL = G * SW
GCH = GEL // LANES
SCALE = -1.0 / float(B * T)


def _mesh():
    return plsc.VectorSubcoreMesh(core_axis_name="c", subcore_axis_name="s")


@functools.partial(
    pl.kernel,
    mesh=_mesh(),
    out_type=jax.ShapeDtypeStruct((NW, LANES), jnp.float32),
    scratch_types=[
        pltpu.VMEM((CHUNK // SW, SW), jnp.int32),
        pltpu.VMEM((CHUNK,), jnp.int32),
        pltpu.VMEM((CHUNK,), jnp.float32),
        pltpu.VMEM((LANES,), jnp.float32),
        pltpu.SemaphoreType.DMA,
        pltpu.SemaphoreType.DMA,
        pltpu.SemaphoreType.DMA,
        pltpu.SemaphoreType.DMA,
        pltpu.SemaphoreType.DMA,
    ],
)
def _sc_gather_loss(tags_hbm, flat_hbm, out_hbm,
                    lbl_v, idx_v, val_v, part_v,
                    sem_a, sem_b, sem_c, sem_d, sem_l):
    nc = 2
    wid = lax.axis_index("s") * nc + lax.axis_index("c")

    def lstage(m, c):
        pltpu.make_async_copy(tags_hbm.at[m, wid],
                              lbl_v.at[pl.ds(m * 8, 8), :], sem_l).start()
        return c

    lax.fori_loop(0, T // 8, lstage, 0)

    def lwait(m, c):
        pltpu.make_async_copy(tags_hbm.at[0, 0],
                              lbl_v.at[pl.ds(0, 8), :], sem_l).wait()
        return c

    lax.fori_loop(0, T // 8, lwait, 0)

    wbase = wid * 1024
    lane = lax.iota(jnp.int32, 16)

    def cbody(i, c):
        sl = pl.ds(i * LANES, LANES)
        q0 = (i & 7) << 4
        v = lbl_v[i >> 3, pl.ds(q0, LANES)]
        idx_v[sl] = ((v >> 3) << 15) + ((v & 7) << 7) + (wbase + q0 + lane)
        return c

    def compute(g):
        lax.fori_loop(g * GCH, (g + 1) * GCH, cbody, 0, unroll=4)

    def fire(g, sem):
        def fbody(t, c):
            sl = pl.ds(t * SW, SW)
            pltpu.make_async_copy(flat_hbm.at[idx_v.at[sl]], val_v.at[sl],
                                  sem).start()
            return c
        lax.fori_loop(g * G, (g + 1) * G, fbody, 0)

    def drain(g, sem):
        pltpu.make_async_copy(flat_hbm.at[pl.ds(0, GEL)],
                              val_v.at[pl.ds(g * GEL, GEL)], sem).wait()

    def rbody(i, acc):
        sl = pl.ds(i * LANES, LANES)
        v = lbl_v[i >> 3, pl.ds((i & 7) << 4, LANES)]
        return acc + jnp.where(v != 0, val_v[sl], 0.0)

    def reduce(g, acc):
        return lax.fori_loop(g * GCH, (g + 1) * GCH, rbody, acc, unroll=4)

    def fused_body(gr, gc):
        def fbody(i, acc):
            slr = pl.ds(gr * GEL + i * LANES, LANES)
            ir = gr * GCH + i
            vr = lbl_v[ir >> 3, pl.ds((ir & 7) << 4, LANES)]
            acc = acc + jnp.where(vr != 0, val_v[slr], 0.0)
            ic = gc * GCH + i
            slc = pl.ds(gc * GEL + i * LANES, LANES)
            q0 = (ic & 7) << 4
            v = lbl_v[ic >> 3, pl.ds(q0, LANES)]
            idx_v[slc] = ((v >> 3) << 15) + ((v & 7) << 7) + (wbase + q0 + lane)
            return acc
        return fbody

    sems = (sem_a, sem_b, sem_c, sem_d, sem_l)
    nsem = len(sems)
    acc = jnp.zeros((LANES,), jnp.float32)
    for g in range(DEPTH):
        compute(g)
        fire(g, sems[g])
    for g in range(NG):
        drain(g, sems[g % nsem])
        if g + DEPTH < NG:
            acc = lax.fori_loop(0, GCH, fused_body(g, g + DEPTH), acc,
                                unroll=4)
            fire(g + DEPTH, sems[(g + DEPTH) % nsem])
        else:
            acc = reduce(g, acc)

    part_v[...] = acc * SCALE
    pltpu.sync_copy(part_v, out_hbm.at[wid])


def kernel(log_prob, tags_label):
    flat = (log_prob.T
            .reshape(V // 8, 8, B // 128, 128)
            .transpose(0, 2, 1, 3)
            .reshape(-1))
    tags = (tags_label.astype(jnp.int32).T
            .reshape(T // 8, 8, B // 128, 128)
            .transpose(0, 2, 1, 3))
    partials = _sc_gather_loss(tags, flat)
    return jnp.sum(partials)
